# Initial kernel scaffold; baseline (speedup 1.0000x reference)
import functools

import jax
import jax.numpy as jnp
from jax import lax
from jax.experimental import pallas as pl
from jax.experimental.pallas import tpu as pltpu
from jax.experimental.pallas import tpu_sc as plsc

N_NODES = 50000
DIM = 64
N_LAYERS = 3
N_EDGES = 800000

NC = 2
NS = 16
HALF = N_NODES // NC
DUMMY = HALF
ACC_ROWS = HALF + 8
RPT = ACC_ROWS // NS
RPT_LAST = HALF - (NS - 1) * RPT
EPT = N_EDGES // NS
K = 80
NCHUNK = EPT // K

_mesh = plsc.VectorSubcoreMesh(core_axis_name="c", subcore_axis_name="s")


def _layer_body(x, row, col, w, zeros, out, acc, ridx, cval, cidx, wv, rows, sem):
    c = lax.axis_index("c")
    s = lax.axis_index("s")

    pltpu.sync_copy(zeros.at[pl.ds(0, RPT)], acc.at[pl.ds(s * RPT, RPT)])
    plsc.subcore_barrier()

    def chunk(k, carry):
        e0 = s * EPT + k * K
        pltpu.sync_copy(row.at[pl.ds(e0, K)], ridx)
        pltpu.sync_copy(col.at[pl.ds(e0, K)], cval)
        pltpu.sync_copy(w.at[pl.ds(e0, K)], wv)

        lo = c * HALF
        for g in range(K // 16):
            cv = cval[pl.ds(g * 16, 16)]
            loc = cv - lo
            ok = (loc >= 0) & (loc < HALF)
            cidx[pl.ds(g * 16, 16)] = jnp.where(ok, loc, DUMMY)

        pltpu.async_copy(x.at[ridx], rows, sem).wait()

        def edge(e, ecarry):
            wb = plsc.load_gather(wv, [jnp.full((16,), e, jnp.int32)])
            for g in range(DIM // 16):
                rows[e, pl.ds(g * 16, 16)] = rows[e, pl.ds(g * 16, 16)] * wb
            return ecarry

        lax.fori_loop(0, K, edge, 0)

        pltpu.sync_copy(rows, acc.at[cidx], add=True)
        return carry

    lax.fori_loop(0, NCHUNK, chunk, 0)
    plsc.subcore_barrier()

    base = s * RPT
    obase = c * HALF + s * RPT

    @pl.when(s < NS - 1)
    def _():
        pltpu.sync_copy(acc.at[pl.ds(base, RPT)], out.at[pl.ds(obase, RPT)])

    @pl.when(s == NS - 1)
    def _():
        pltpu.sync_copy(acc.at[pl.ds(base, RPT_LAST)], out.at[pl.ds(obase, RPT_LAST)])


_layer = pl.kernel(
    _layer_body,
    out_type=jax.ShapeDtypeStruct((N_NODES, DIM), jnp.float32),
    mesh=_mesh,
    scratch_types=[
        pltpu.VMEM_SHARED((ACC_ROWS, DIM), jnp.float32),
        pltpu.VMEM((K,), jnp.int32),
        pltpu.VMEM((K,), jnp.int32),
        pltpu.VMEM((K,), jnp.int32),
        pltpu.VMEM((K,), jnp.float32),
        pltpu.VMEM((K, DIM), jnp.float32),
        pltpu.SemaphoreType.DMA,
    ],
)

MTOT = N_NODES * DIM
MEPT = MTOT // (NC * NS)
MC = 10000
MNCH = MEPT // MC


def _mean_body(a0, a1, a2, a3, o, b0, b1, b2, b3, ob):
    c = lax.axis_index("c")
    s = lax.axis_index("s")
    base = (s * NC + c) * MEPT

    def chunk(j, carry):
        off = base + j * MC
        pltpu.sync_copy(a0.at[pl.ds(off, MC)], b0)
        pltpu.sync_copy(a1.at[pl.ds(off, MC)], b1)
        pltpu.sync_copy(a2.at[pl.ds(off, MC)], b2)
        pltpu.sync_copy(a3.at[pl.ds(off, MC)], b3)

        def grp(g, gcarry):
            p = g * 16
            ob[pl.ds(p, 16)] = (
                b0[pl.ds(p, 16)] + b1[pl.ds(p, 16)] + b2[pl.ds(p, 16)] + b3[pl.ds(p, 16)]
            ) * 0.25
            return gcarry

        lax.fori_loop(0, MC // 16, grp, 0)
        pltpu.sync_copy(ob, o.at[pl.ds(off, MC)])
        return carry

    lax.fori_loop(0, MNCH, chunk, 0)


_mean = pl.kernel(
    _mean_body,
    out_type=jax.ShapeDtypeStruct((MTOT,), jnp.float32),
    mesh=_mesh,
    scratch_types=[
        pltpu.VMEM((MC,), jnp.float32),
        pltpu.VMEM((MC,), jnp.float32),
        pltpu.VMEM((MC,), jnp.float32),
        pltpu.VMEM((MC,), jnp.float32),
        pltpu.VMEM((MC,), jnp.float32),
    ],
)


def kernel(embedding, edge_index, edge_weight):
    row = edge_index[0]
    col = edge_index[1]
    zeros = jnp.zeros((RPT + 5, DIM), jnp.float32)
    x0 = embedding
    x1 = _layer(x0, row, col, edge_weight, zeros)
    x2 = _layer(x1, row, col, edge_weight, zeros)
    x3 = _layer(x2, row, col, edge_weight, zeros)
    of = _mean(x0.reshape(-1), x1.reshape(-1), x2.reshape(-1), x3.reshape(-1))
    return of.reshape(N_NODES, DIM)


# speedup vs baseline: 2.0060x; 2.0060x over previous
import functools

import jax
import jax.numpy as jnp
from jax import lax
from jax.experimental import pallas as pl
from jax.experimental.pallas import tpu as pltpu
from jax.experimental.pallas import tpu_sc as plsc

N_NODES = 50000
DIM = 64
N_LAYERS = 3
N_EDGES = 800000

NC = 2
NS = 16
HALF = N_NODES // NC
DUMMY = HALF
RPT = 1568
ACC_ROWS = RPT * NS
RPT_LAST = HALF - (NS - 1) * RPT
EPT = N_EDGES // NS
K = 80
NCHUNK = EPT // K

_mesh = plsc.VectorSubcoreMesh(core_axis_name="c", subcore_axis_name="s")


def _layer_body(x, row, col, w, zeros, out, acc, ridx, cval, cidx, wv, rows, sem):
    c = lax.axis_index("c")
    s = lax.axis_index("s")

    pltpu.sync_copy(zeros, acc.at[pl.ds(s * RPT, RPT)])
    plsc.subcore_barrier()

    def chunk(k, carry):
        e0 = s * EPT + k * K
        pltpu.sync_copy(row.at[pl.ds(e0, K)], ridx)
        pltpu.sync_copy(col.at[pl.ds(e0, K)], cval)
        pltpu.sync_copy(w.at[pl.ds(e0, K)], wv.at[pl.ds(0, K)])

        lo = c * HALF
        for g in range(K // 16):
            cv = cval[pl.ds(g * 16, 16)]
            loc = cv - lo
            ok = (loc >= 0) & (loc < HALF)
            cidx[pl.ds(g * 16, 16)] = jnp.where(ok, loc, DUMMY)

        pltpu.async_copy(x.at[ridx], rows, sem).wait()

        def edge(e, ecarry):
            wb = jnp.full((16,), wv[pl.ds(e, 16)][0], jnp.float32)
            for g in range(DIM // 16):
                rows[e, pl.ds(g * 16, 16)] = rows[e, pl.ds(g * 16, 16)] * wb
            return ecarry

        lax.fori_loop(0, K, edge, 0)

        pltpu.sync_copy(rows, acc.at[cidx], add=True)
        return carry

    lax.fori_loop(0, NCHUNK, chunk, 0)
    plsc.subcore_barrier()

    base = s * RPT
    obase = c * HALF + s * RPT

    @pl.when(s < NS - 1)
    def _():
        pltpu.sync_copy(acc.at[pl.ds(base, RPT)], out.at[pl.ds(obase, RPT)])

    @pl.when(s == NS - 1)
    def _():
        pltpu.sync_copy(acc.at[pl.ds(base, RPT_LAST)], out.at[pl.ds(obase, RPT_LAST)])


_layer = pl.kernel(
    _layer_body,
    out_type=jax.ShapeDtypeStruct((N_NODES, DIM), jnp.float32),
    mesh=_mesh,
    compiler_params=pltpu.CompilerParams(use_tc_tiling_on_sc=False),
    scratch_types=[
        pltpu.VMEM_SHARED((ACC_ROWS, DIM), jnp.float32),
        pltpu.VMEM((K,), jnp.int32),
        pltpu.VMEM((K,), jnp.int32),
        pltpu.VMEM((K,), jnp.int32),
        pltpu.VMEM((K + 16,), jnp.float32),
        pltpu.VMEM((K, DIM), jnp.float32),
        pltpu.SemaphoreType.DMA,
    ],
)

MTOT = N_NODES * DIM
MEPT = MTOT // (NC * NS)
MC = 10000
MNCH = MEPT // MC


def _mean_body(a0, a1, a2, a3, o, b0, b1, b2, b3, ob):
    c = lax.axis_index("c")
    s = lax.axis_index("s")
    base = (s * NC + c) * MEPT

    def chunk(j, carry):
        off = base + j * MC
        pltpu.sync_copy(a0.at[pl.ds(off, MC)], b0)
        pltpu.sync_copy(a1.at[pl.ds(off, MC)], b1)
        pltpu.sync_copy(a2.at[pl.ds(off, MC)], b2)
        pltpu.sync_copy(a3.at[pl.ds(off, MC)], b3)

        def grp(g, gcarry):
            p = g * 16
            ob[pl.ds(p, 16)] = (
                b0[pl.ds(p, 16)] + b1[pl.ds(p, 16)] + b2[pl.ds(p, 16)] + b3[pl.ds(p, 16)]
            ) * 0.25
            return gcarry

        lax.fori_loop(0, MC // 16, grp, 0)
        pltpu.sync_copy(ob, o.at[pl.ds(off, MC)])
        return carry

    lax.fori_loop(0, MNCH, chunk, 0)


_mean = pl.kernel(
    _mean_body,
    out_type=jax.ShapeDtypeStruct((MTOT,), jnp.float32),
    mesh=_mesh,
    scratch_types=[
        pltpu.VMEM((MC,), jnp.float32),
        pltpu.VMEM((MC,), jnp.float32),
        pltpu.VMEM((MC,), jnp.float32),
        pltpu.VMEM((MC,), jnp.float32),
        pltpu.VMEM((MC,), jnp.float32),
    ],
)


def kernel(embedding, edge_index, edge_weight):
    row = edge_index[0]
    col = edge_index[1]
    zeros = jnp.zeros((RPT, DIM), jnp.float32)
    x0 = embedding
    x1 = _layer(x0, row, col, edge_weight, zeros)
    x2 = _layer(x1, row, col, edge_weight, zeros)
    x3 = _layer(x2, row, col, edge_weight, zeros)
    of = _mean(x0.reshape(-1), x1.reshape(-1), x2.reshape(-1), x3.reshape(-1))
    return of.reshape(N_NODES, DIM)


# speedup vs baseline: 6.2872x; 3.1342x over previous
import functools

import jax
import jax.numpy as jnp
from jax import lax
from jax.experimental import pallas as pl
from jax.experimental.pallas import tpu as pltpu
from jax.experimental.pallas import tpu_sc as plsc

N_NODES = 50000
DIM = 64
N_LAYERS = 3
N_EDGES = 800000

NC = 2
NS = 16
HALF = N_NODES // NC
DUMMY = HALF
RPT = 1568
ACC_ROWS = RPT * NS
RPT_LAST = HALF - (NS - 1) * RPT
EPT = N_EDGES // NS
K = 80
NCHUNK = EPT // K
NBUF = 5

_mesh = plsc.VectorSubcoreMesh(core_axis_name="c", subcore_axis_name="s")


def _layer_body(x, row, col, w, zeros, out,
                acc, ridx, cval, cidx, wv, rows, esem, gsem, ssem):
    c = lax.axis_index("c")
    s = lax.axis_index("s")

    pltpu.sync_copy(zeros, acc.at[pl.ds(s * RPT, RPT)])
    plsc.subcore_barrier()

    ebase = s * EPT
    lo = c * HALF

    def issue_edges(k, b):
        e0 = ebase + k * K
        pltpu.async_copy(row.at[pl.ds(e0, K)], ridx.at[b], esem.at[b])
        pltpu.async_copy(col.at[pl.ds(e0, K)], cval.at[b], esem.at[b])
        pltpu.async_copy(w.at[pl.ds(e0, K)], wv.at[b, pl.ds(0, K)], esem.at[b])

    def wait_edges(b):
        pltpu.make_async_copy(row.at[pl.ds(0, K)], ridx.at[b], esem.at[b]).wait()
        pltpu.make_async_copy(col.at[pl.ds(0, K)], cval.at[b], esem.at[b]).wait()
        pltpu.make_async_copy(w.at[pl.ds(0, K)], wv.at[b, pl.ds(0, K)], esem.at[b]).wait()

    def issue_gather(b):
        for g in range(K // 16):
            cv = cval[b, pl.ds(g * 16, 16)]
            loc = cv - lo
            ok = (loc >= 0) & (loc < HALF)
            cidx[b, pl.ds(g * 16, 16)] = jnp.where(ok, loc, DUMMY)
        pltpu.async_copy(x.at[ridx.at[b]], rows.at[b], gsem.at[b])

    def wait_gather(b):
        pltpu.make_async_copy(x.at[ridx.at[b]], rows.at[b], gsem.at[b]).wait()

    def scale_and_scatter(b):
        def edge(e, ecarry):
            wb = jnp.full((16,), wv[b, pl.ds(e, 16)][0], jnp.float32)
            for g in range(DIM // 16):
                rows[b, e, pl.ds(g * 16, 16)] = rows[b, e, pl.ds(g * 16, 16)] * wb
            return ecarry

        lax.fori_loop(0, K, edge, 0, unroll=4)
        pltpu.async_copy(rows.at[b], acc.at[cidx.at[b]], ssem.at[b], add=True)

    def wait_scatter(b):
        pltpu.make_async_copy(rows.at[b], acc.at[cidx.at[b]], ssem.at[b]).wait()

    issue_edges(0, 0)
    issue_edges(1, 1)
    wait_edges(0)
    issue_gather(0)
    for b in range(NBUF):
        if b + 2 >= NBUF:
            wait_scatter((b + 2) % NBUF)
        issue_edges(b + 2, (b + 2) % NBUF)
        wait_edges((b + 1) % NBUF)
        issue_gather((b + 1) % NBUF)
        wait_gather(b)
        scale_and_scatter(b)

    def steady(j, carry):
        k0 = j * NBUF
        for b in range(NBUF):
            wait_scatter((b + 2) % NBUF)
            issue_edges(k0 + b + 2, (b + 2) % NBUF)
            wait_edges((b + 1) % NBUF)
            issue_gather((b + 1) % NBUF)
            wait_gather(b)
            scale_and_scatter(b)
        return carry

    lax.fori_loop(1, NCHUNK // NBUF - 1, steady, 0)

    k0 = NCHUNK - NBUF
    for b in range(NBUF):
        if b + 2 < NBUF:
            wait_scatter((b + 2) % NBUF)
            issue_edges(k0 + b + 2, (b + 2) % NBUF)
        if b + 1 < NBUF:
            wait_edges((b + 1) % NBUF)
            issue_gather((b + 1) % NBUF)
        wait_gather(b)
        scale_and_scatter(b)
    for b in range(NBUF):
        wait_scatter(b)

    plsc.subcore_barrier()

    base = s * RPT
    obase = c * HALF + s * RPT

    @pl.when(s < NS - 1)
    def _():
        pltpu.sync_copy(acc.at[pl.ds(base, RPT)], out.at[pl.ds(obase, RPT)])

    @pl.when(s == NS - 1)
    def _():
        pltpu.sync_copy(acc.at[pl.ds(base, RPT_LAST)], out.at[pl.ds(obase, RPT_LAST)])


_layer = pl.kernel(
    _layer_body,
    out_type=jax.ShapeDtypeStruct((N_NODES, DIM), jnp.float32),
    mesh=_mesh,
    compiler_params=pltpu.CompilerParams(use_tc_tiling_on_sc=False),
    scratch_types=[
        pltpu.VMEM_SHARED((ACC_ROWS, DIM), jnp.float32),
        pltpu.VMEM((NBUF, K), jnp.int32),
        pltpu.VMEM((NBUF, K), jnp.int32),
        pltpu.VMEM((NBUF, K), jnp.int32),
        pltpu.VMEM((NBUF, K + 16), jnp.float32),
        pltpu.VMEM((NBUF, K, DIM), jnp.float32),
        pltpu.SemaphoreType.DMA((NBUF,)),
        pltpu.SemaphoreType.DMA((NBUF,)),
        pltpu.SemaphoreType.DMA((NBUF,)),
    ],
)

MTOT = N_NODES * DIM
MEPT = MTOT // (NC * NS)
MC = 10000
MNCH = MEPT // MC


def _mean_body(a0, a1, a2, a3, o, b0, b1, b2, b3, ob):
    c = lax.axis_index("c")
    s = lax.axis_index("s")
    base = (s * NC + c) * MEPT

    def chunk(j, carry):
        off = base + j * MC
        pltpu.sync_copy(a0.at[pl.ds(off, MC)], b0)
        pltpu.sync_copy(a1.at[pl.ds(off, MC)], b1)
        pltpu.sync_copy(a2.at[pl.ds(off, MC)], b2)
        pltpu.sync_copy(a3.at[pl.ds(off, MC)], b3)

        def grp(g, gcarry):
            p = g * 16
            ob[pl.ds(p, 16)] = (
                b0[pl.ds(p, 16)] + b1[pl.ds(p, 16)] + b2[pl.ds(p, 16)] + b3[pl.ds(p, 16)]
            ) * 0.25
            return gcarry

        lax.fori_loop(0, MC // 16, grp, 0)
        pltpu.sync_copy(ob, o.at[pl.ds(off, MC)])
        return carry

    lax.fori_loop(0, MNCH, chunk, 0)


_mean = pl.kernel(
    _mean_body,
    out_type=jax.ShapeDtypeStruct((MTOT,), jnp.float32),
    mesh=_mesh,
    scratch_types=[
        pltpu.VMEM((MC,), jnp.float32),
        pltpu.VMEM((MC,), jnp.float32),
        pltpu.VMEM((MC,), jnp.float32),
        pltpu.VMEM((MC,), jnp.float32),
        pltpu.VMEM((MC,), jnp.float32),
    ],
)


def kernel(embedding, edge_index, edge_weight):
    row = edge_index[0]
    col = edge_index[1]
    zeros = jnp.zeros((RPT, DIM), jnp.float32)
    x0 = embedding
    x1 = _layer(x0, row, col, edge_weight, zeros)
    x2 = _layer(x1, row, col, edge_weight, zeros)
    x3 = _layer(x2, row, col, edge_weight, zeros)
    of = _mean(x0.reshape(-1), x1.reshape(-1), x2.reshape(-1), x3.reshape(-1))
    return of.reshape(N_NODES, DIM)


# speedup vs baseline: 6.3200x; 1.0052x over previous
import functools

import jax
import jax.numpy as jnp
from jax import lax
from jax.experimental import pallas as pl
from jax.experimental.pallas import tpu as pltpu
from jax.experimental.pallas import tpu_sc as plsc

N_NODES = 50000
DIM = 64
N_LAYERS = 3
N_EDGES = 800000

NC = 2
NS = 16
HALF = N_NODES // NC
DUMMY = HALF
RPT = 1568
ACC_ROWS = RPT * NS
RPT_LAST = HALF - (NS - 1) * RPT
EPT = N_EDGES // NS
K = 80
NCHUNK = EPT // K
NBUF = 5

_mesh = plsc.VectorSubcoreMesh(core_axis_name="c", subcore_axis_name="s")


def _layer_body(x, row, col, w, zeros, out,
                acc, ridx, cval, cidx, wv, rows, esem, gsem, ssem):
    c = lax.axis_index("c")
    s = lax.axis_index("s")

    pltpu.sync_copy(zeros, acc.at[pl.ds(s * RPT, RPT)])
    plsc.subcore_barrier()

    ebase = s * EPT
    lo = c * HALF

    def issue_edges(k, b):
        e0 = ebase + k * K
        pltpu.async_copy(row.at[pl.ds(e0, K)], ridx.at[b], esem.at[b])
        pltpu.async_copy(col.at[pl.ds(e0, K)], cval.at[b], esem.at[b])
        pltpu.async_copy(w.at[pl.ds(e0, K)], wv.at[b, pl.ds(0, K)], esem.at[b])

    def wait_edges(b):
        pltpu.make_async_copy(row.at[pl.ds(0, K)], ridx.at[b], esem.at[b]).wait()
        pltpu.make_async_copy(col.at[pl.ds(0, K)], cval.at[b], esem.at[b]).wait()
        pltpu.make_async_copy(w.at[pl.ds(0, K)], wv.at[b, pl.ds(0, K)], esem.at[b]).wait()

    def issue_gather(b):
        for g in range(K // 16):
            cv = cval[b, pl.ds(g * 16, 16)]
            loc = cv - lo
            ok = (loc >= 0) & (loc < HALF)
            cidx[b, pl.ds(g * 16, 16)] = jnp.where(ok, loc, DUMMY)
        pltpu.async_copy(x.at[ridx.at[b]], rows.at[b], gsem.at[b])

    def wait_gather(b):
        pltpu.make_async_copy(x.at[ridx.at[b]], rows.at[b], gsem.at[b]).wait()

    def scale_and_scatter(b):
        @plsc.parallel_loop(0, K, unroll=8)
        def _scale(e):
            wb = jnp.full((16,), wv[b, pl.ds(e, 16)][0], jnp.float32)
            for d in range(DIM // 16):
                rows[b, e, pl.ds(d * 16, 16)] = rows[b, e, pl.ds(d * 16, 16)] * wb
        pltpu.async_copy(rows.at[b], acc.at[cidx.at[b]], ssem.at[b], add=True)

    def wait_scatter(b):
        pltpu.make_async_copy(rows.at[b], acc.at[cidx.at[b]], ssem.at[b]).wait()

    issue_edges(0, 0)
    issue_edges(1, 1)
    wait_edges(0)
    issue_gather(0)
    for b in range(NBUF):
        if b + 2 >= NBUF:
            wait_scatter((b + 2) % NBUF)
        issue_edges(b + 2, (b + 2) % NBUF)
        wait_edges((b + 1) % NBUF)
        issue_gather((b + 1) % NBUF)
        wait_gather(b)
        scale_and_scatter(b)

    def steady(j, carry):
        k0 = j * NBUF
        for b in range(NBUF):
            wait_scatter((b + 2) % NBUF)
            issue_edges(k0 + b + 2, (b + 2) % NBUF)
            wait_edges((b + 1) % NBUF)
            issue_gather((b + 1) % NBUF)
            wait_gather(b)
            scale_and_scatter(b)
        return carry

    lax.fori_loop(1, NCHUNK // NBUF - 1, steady, 0)

    k0 = NCHUNK - NBUF
    for b in range(NBUF):
        if b + 2 < NBUF:
            wait_scatter((b + 2) % NBUF)
            issue_edges(k0 + b + 2, (b + 2) % NBUF)
        if b + 1 < NBUF:
            wait_edges((b + 1) % NBUF)
            issue_gather((b + 1) % NBUF)
        wait_gather(b)
        scale_and_scatter(b)
    for b in range(NBUF):
        wait_scatter(b)

    plsc.subcore_barrier()

    base = s * RPT
    obase = c * HALF + s * RPT

    @pl.when(s < NS - 1)
    def _():
        pltpu.sync_copy(acc.at[pl.ds(base, RPT)], out.at[pl.ds(obase, RPT)])

    @pl.when(s == NS - 1)
    def _():
        pltpu.sync_copy(acc.at[pl.ds(base, RPT_LAST)], out.at[pl.ds(obase, RPT_LAST)])


_layer = pl.kernel(
    _layer_body,
    out_type=jax.ShapeDtypeStruct((N_NODES, DIM), jnp.float32),
    mesh=_mesh,
    compiler_params=pltpu.CompilerParams(use_tc_tiling_on_sc=False),
    scratch_types=[
        pltpu.VMEM_SHARED((ACC_ROWS, DIM), jnp.float32),
        pltpu.VMEM((NBUF, K), jnp.int32),
        pltpu.VMEM((NBUF, K), jnp.int32),
        pltpu.VMEM((NBUF, K), jnp.int32),
        pltpu.VMEM((NBUF, K + 16), jnp.float32),
        pltpu.VMEM((NBUF, K, DIM), jnp.float32),
        pltpu.SemaphoreType.DMA((NBUF,)),
        pltpu.SemaphoreType.DMA((NBUF,)),
        pltpu.SemaphoreType.DMA((NBUF,)),
    ],
)

MTOT = N_NODES * DIM
MEPT = MTOT // (NC * NS)
MC = 10000
MNCH = MEPT // MC


def _mean_body(a0, a1, a2, a3, o, b0, b1, b2, b3, ob):
    c = lax.axis_index("c")
    s = lax.axis_index("s")
    base = (s * NC + c) * MEPT

    def chunk(j, carry):
        off = base + j * MC
        pltpu.sync_copy(a0.at[pl.ds(off, MC)], b0)
        pltpu.sync_copy(a1.at[pl.ds(off, MC)], b1)
        pltpu.sync_copy(a2.at[pl.ds(off, MC)], b2)
        pltpu.sync_copy(a3.at[pl.ds(off, MC)], b3)

        def grp(g, gcarry):
            p = g * 16
            ob[pl.ds(p, 16)] = (
                b0[pl.ds(p, 16)] + b1[pl.ds(p, 16)] + b2[pl.ds(p, 16)] + b3[pl.ds(p, 16)]
            ) * 0.25
            return gcarry

        lax.fori_loop(0, MC // 16, grp, 0)
        pltpu.sync_copy(ob, o.at[pl.ds(off, MC)])
        return carry

    lax.fori_loop(0, MNCH, chunk, 0)


_mean = pl.kernel(
    _mean_body,
    out_type=jax.ShapeDtypeStruct((MTOT,), jnp.float32),
    mesh=_mesh,
    scratch_types=[
        pltpu.VMEM((MC,), jnp.float32),
        pltpu.VMEM((MC,), jnp.float32),
        pltpu.VMEM((MC,), jnp.float32),
        pltpu.VMEM((MC,), jnp.float32),
        pltpu.VMEM((MC,), jnp.float32),
    ],
)


def kernel(embedding, edge_index, edge_weight):
    row = edge_index[0]
    col = edge_index[1]
    zeros = jnp.zeros((RPT, DIM), jnp.float32)
    x0 = embedding
    x1 = _layer(x0, row, col, edge_weight, zeros)
    x2 = _layer(x1, row, col, edge_weight, zeros)
    x3 = _layer(x2, row, col, edge_weight, zeros)
    of = _mean(x0.reshape(-1), x1.reshape(-1), x2.reshape(-1), x3.reshape(-1))
    return of.reshape(N_NODES, DIM)


# speedup vs baseline: 12.0430x; 1.9055x over previous
import functools

import jax
import jax.numpy as jnp
from jax import lax
from jax.experimental import pallas as pl
from jax.experimental.pallas import tpu as pltpu
from jax.experimental.pallas import tpu_sc as plsc

N_NODES = 50000
DIM = 64
N_EDGES = 800000

NC = 2
NS = 16
NW = NC * NS
HALF = N_NODES // NC
DUMMY = HALF
RPT = 1568
ACC_ROWS = RPT * NS
RPT_LAST = HALF - (NS - 1) * RPT
K = 80
NBUF = 5
PEPT = N_EDGES // NW
SEGCAP = 25040
BIN = 1000
NBLK = PEPT // BIN
STCAP = 1088
FLUSH = 960

_mesh = plsc.VectorSubcoreMesh(core_axis_name="c", subcore_axis_name="s")

_IOTA = None


def _partition_body(row, col, w, rowp, colp, wp, ncnk,
                    inr, inc, inw, st0r, st0c, st0w, st1r, st1c, st1w, cbuf):
    c = lax.axis_index("c")
    s = lax.axis_index("s")
    t = c * NS + s
    ebase = t * PEPT

    iota = lax.iota(jnp.int32, 16)

    def seg_off(bkt):
        return (bkt * NW + t) * SEGCAP

    def flush_if_full(bkt, sr, sc, sw, cnt, nch):
        def do_flush(args):
            cnt, nch = args
            off = seg_off(bkt) + nch * K
            pltpu.sync_copy(sr.at[pl.ds(0, FLUSH)], rowp.at[pl.ds(off, FLUSH)])
            pltpu.sync_copy(sc.at[pl.ds(0, FLUSH)], colp.at[pl.ds(off, FLUSH)])
            pltpu.sync_copy(sw.at[pl.ds(0, FLUSH)], wp.at[pl.ds(off, FLUSH)])
            tr = sr[pl.ds(FLUSH, 16)]
            tc_ = sc[pl.ds(FLUSH, 16)]
            tw = sw[pl.ds(FLUSH, 16)]
            sr[pl.ds(0, 16)] = tr
            sc[pl.ds(0, 16)] = tc_
            sw[pl.ds(0, 16)] = tw
            return cnt - FLUSH, nch + FLUSH // K

        return lax.cond(cnt >= FLUSH, do_flush, lambda a: a, (cnt, nch))

    def block(j, carry):
        cnt0, nch0, cnt1, nch1 = carry
        off = ebase + j * BIN
        pltpu.sync_copy(row.at[pl.ds(off, BIN)], inr)
        pltpu.sync_copy(col.at[pl.ds(off, BIN)], inc)
        pltpu.sync_copy(w.at[pl.ds(off, BIN)], inw)

        def grp(g, gc):
            cnt0, nch0, cnt1, nch1 = gc
            p = g * 16
            valid = jnp.where(g < BIN // 16, 16, BIN - (BIN // 16) * 16)
            vm = iota < valid
            rv = inr[pl.ds(p, 16)]
            cv = inc[pl.ds(p, 16)]
            wvv = inw[pl.ds(p, 16)]

            m0 = vm & (cv < HALF)
            cs0 = plsc.cumsum(m0.astype(jnp.int32))
            i0 = jnp.where(m0, cnt0 + cs0 - 1, STCAP + iota)
            plsc.store_scatter(st0r, [i0], rv)
            plsc.store_scatter(st0c, [i0], cv)
            plsc.store_scatter(st0w, [i0], wvv)
            cnt0 = cnt0 + cs0[15]
            cnt0, nch0 = flush_if_full(0, st0r, st0c, st0w, cnt0, nch0)

            m1 = vm & (cv >= HALF)
            cs1 = plsc.cumsum(m1.astype(jnp.int32))
            i1 = jnp.where(m1, cnt1 + cs1 - 1, STCAP + iota)
            plsc.store_scatter(st1r, [i1], rv)
            plsc.store_scatter(st1c, [i1], cv - HALF)
            plsc.store_scatter(st1w, [i1], wvv)
            cnt1 = cnt1 + cs1[15]
            cnt1, nch1 = flush_if_full(1, st1r, st1c, st1w, cnt1, nch1)
            return (cnt0, nch0, cnt1, nch1)

        ngrp = BIN // 16 + (1 if BIN % 16 else 0)
        return lax.fori_loop(0, ngrp, grp, (cnt0, nch0, cnt1, nch1))

    z = jnp.int32(0)
    cnt0, nch0, cnt1, nch1 = lax.fori_loop(0, NBLK, block, (z, z, z, z))

    dz = jnp.zeros((16,), jnp.int32)
    dzf = jnp.zeros((16,), jnp.float32)
    dd = jnp.full((16,), DUMMY, jnp.int32)

    for bkt in range(2):
        cnt = (cnt0, cnt1)[bkt]
        nch = (nch0, nch1)[bkt]
        sr, sc, sw = ((st0r, st0c, st0w), (st1r, st1c, st1w))[bkt]
        for i in range(6):
            sr[pl.ds(cnt + i * 16, 16)] = dz
            sc[pl.ds(cnt + i * 16, 16)] = dd
            sw[pl.ds(cnt + i * 16, 16)] = dzf

        def tail(i, a, bkt=bkt, cnt=cnt, nch=nch, sr=sr, sc=sc, sw=sw):
            nf = a

            @pl.when(i * K < cnt)
            def _():
                off = seg_off(bkt) + (nch + i) * K
                pltpu.sync_copy(sr.at[pl.ds(i * K, K)], rowp.at[pl.ds(off, K)])
                pltpu.sync_copy(sc.at[pl.ds(i * K, K)], colp.at[pl.ds(off, K)])
                pltpu.sync_copy(sw.at[pl.ds(i * K, K)], wp.at[pl.ds(off, K)])

            return nf + jnp.where(i * K < cnt, 1, 0)

        nf = lax.fori_loop(0, STCAP // K + 1, tail, z)
        n = nch + nf
        nv = jnp.full((16,), n, jnp.int32)
        jm = ((nv.astype(jnp.float32) + 8.5) * 0.2).astype(jnp.int32)
        cbuf[pl.ds(0, 16)] = jnp.where(iota == 0, nv, jnp.where(iota == 1, jm, 0))
        pltpu.sync_copy(cbuf.at[pl.ds(0, 8)], ncnk.at[pl.ds((bkt * NW + t) * 8, 8)])


_partition = pl.kernel(
    _partition_body,
    out_type=(
        jax.ShapeDtypeStruct((2 * NW * SEGCAP,), jnp.int32),
        jax.ShapeDtypeStruct((2 * NW * SEGCAP,), jnp.int32),
        jax.ShapeDtypeStruct((2 * NW * SEGCAP,), jnp.float32),
        jax.ShapeDtypeStruct((2 * NW * 8,), jnp.int32),
    ),
    mesh=_mesh,
    compiler_params=pltpu.CompilerParams(use_tc_tiling_on_sc=False, needs_layout_passes=False),
    scratch_types=[
        pltpu.VMEM((BIN,), jnp.int32),
        pltpu.VMEM((BIN,), jnp.int32),
        pltpu.VMEM((BIN,), jnp.float32),
        pltpu.VMEM((STCAP + 16,), jnp.int32),
        pltpu.VMEM((STCAP + 16,), jnp.int32),
        pltpu.VMEM((STCAP + 16,), jnp.float32),
        pltpu.VMEM((STCAP + 16,), jnp.int32),
        pltpu.VMEM((STCAP + 16,), jnp.int32),
        pltpu.VMEM((STCAP + 16,), jnp.float32),
        pltpu.VMEM((16,), jnp.int32),
    ],
)


def _layer_body(x, rowp, colp, wp, ncnk, zeros, out,
                acc, ridx, cidx, wv, rows, cbuf, esem, gsem, ssem):
    c = lax.axis_index("c")
    s = lax.axis_index("s")

    pltpu.sync_copy(zeros, acc.at[pl.ds(s * RPT, RPT)])

    pltpu.sync_copy(ncnk.at[pl.ds((c * NW + 2 * s) * 8, 8)], cbuf.at[pl.ds(0, 8)])
    pltpu.sync_copy(ncnk.at[pl.ds((c * NW + 2 * s + 1) * 8, 8)], cbuf.at[pl.ds(8, 8)])
    cv = cbuf[pl.ds(0, 16)]
    n0, jm0, n1, jm1 = cv[0], cv[1], cv[8], cv[9]

    plsc.subcore_barrier()

    def process_segment(t, n, jm):
        ebase = (c * NW + t) * SEGCAP

        def issue_edges(k, b):
            e0 = ebase + k * K
            pltpu.async_copy(rowp.at[pl.ds(e0, K)], ridx.at[b], esem.at[b])
            pltpu.async_copy(colp.at[pl.ds(e0, K)], cidx.at[b], esem.at[b])
            pltpu.async_copy(wp.at[pl.ds(e0, K)], wv.at[b, pl.ds(0, K)], esem.at[b])

        def wait_edges(b):
            pltpu.make_async_copy(rowp.at[pl.ds(0, K)], ridx.at[b], esem.at[b]).wait()
            pltpu.make_async_copy(colp.at[pl.ds(0, K)], cidx.at[b], esem.at[b]).wait()
            pltpu.make_async_copy(wp.at[pl.ds(0, K)], wv.at[b, pl.ds(0, K)], esem.at[b]).wait()

        def issue_gather(b):
            pltpu.async_copy(x.at[ridx.at[b]], rows.at[b], gsem.at[b])

        def wait_gather(b):
            pltpu.make_async_copy(x.at[ridx.at[b]], rows.at[b], gsem.at[b]).wait()

        def scale_and_scatter(b):
            @plsc.parallel_loop(0, K, unroll=8)
            def _scale(e):
                wb = jnp.full((16,), wv[b, pl.ds(e, 16)][0], jnp.float32)
                for d in range(DIM // 16):
                    rows[b, e, pl.ds(d * 16, 16)] = rows[b, e, pl.ds(d * 16, 16)] * wb

            pltpu.async_copy(rows.at[b], acc.at[cidx.at[b]], ssem.at[b], add=True)

        def wait_scatter(b):
            pltpu.make_async_copy(rows.at[b], acc.at[cidx.at[b]], ssem.at[b]).wait()

        def virt(j, carry):
            i0 = j * NBUF
            for b in range(NBUF):
                i = i0 + b
                ke = i
                kg = i - 2
                kp = i - 4

                @pl.when((ke >= NBUF) & (ke < n))
                def _():
                    wait_scatter(b)

                @pl.when(ke < n)
                def _():
                    issue_edges(ke, b)

                @pl.when((kg >= 0) & (kg < n))
                def _():
                    wait_edges((b + 3) % NBUF)
                    issue_gather((b + 3) % NBUF)

                @pl.when((kp >= 0) & (kp < n))
                def _():
                    wait_gather((b + 1) % NBUF)
                    scale_and_scatter((b + 1) % NBUF)

            return carry

        lax.fori_loop(0, jm, virt, 0)
        for b in range(NBUF):
            @pl.when(b < n)
            def _():
                wait_scatter(b)

    process_segment(2 * s, n0, jm0)
    process_segment(2 * s + 1, n1, jm1)

    plsc.subcore_barrier()

    base = s * RPT
    obase = c * HALF + s * RPT

    @pl.when(s < NS - 1)
    def _():
        pltpu.sync_copy(acc.at[pl.ds(base, RPT)], out.at[pl.ds(obase, RPT)])

    @pl.when(s == NS - 1)
    def _():
        pltpu.sync_copy(acc.at[pl.ds(base, RPT_LAST)], out.at[pl.ds(obase, RPT_LAST)])


_layer = pl.kernel(
    _layer_body,
    out_type=jax.ShapeDtypeStruct((N_NODES, DIM), jnp.float32),
    mesh=_mesh,
    compiler_params=pltpu.CompilerParams(use_tc_tiling_on_sc=False, needs_layout_passes=False),
    scratch_types=[
        pltpu.VMEM_SHARED((ACC_ROWS, DIM), jnp.float32),
        pltpu.VMEM((NBUF, K), jnp.int32),
        pltpu.VMEM((NBUF, K), jnp.int32),
        pltpu.VMEM((NBUF, K + 16), jnp.float32),
        pltpu.VMEM((NBUF, K, DIM), jnp.float32),
        pltpu.VMEM((16,), jnp.int32),
        pltpu.SemaphoreType.DMA((NBUF,)),
        pltpu.SemaphoreType.DMA((NBUF,)),
        pltpu.SemaphoreType.DMA((NBUF,)),
    ],
)

MTOT = N_NODES * DIM
MEPT = MTOT // (NC * NS)
MC = 10000
MNCH = MEPT // MC


def _mean_body(a0, a1, a2, a3, o, b0, b1, b2, b3, ob):
    c = lax.axis_index("c")
    s = lax.axis_index("s")
    base = (s * NC + c) * MEPT

    def chunk(j, carry):
        off = base + j * MC
        pltpu.sync_copy(a0.at[pl.ds(off, MC)], b0)
        pltpu.sync_copy(a1.at[pl.ds(off, MC)], b1)
        pltpu.sync_copy(a2.at[pl.ds(off, MC)], b2)
        pltpu.sync_copy(a3.at[pl.ds(off, MC)], b3)

        def grp(g, gcarry):
            p = g * 16
            ob[pl.ds(p, 16)] = (
                b0[pl.ds(p, 16)] + b1[pl.ds(p, 16)] + b2[pl.ds(p, 16)] + b3[pl.ds(p, 16)]
            ) * 0.25
            return gcarry

        lax.fori_loop(0, MC // 16, grp, 0)
        pltpu.sync_copy(ob, o.at[pl.ds(off, MC)])
        return carry

    lax.fori_loop(0, MNCH, chunk, 0)


_mean = pl.kernel(
    _mean_body,
    out_type=jax.ShapeDtypeStruct((MTOT,), jnp.float32),
    mesh=_mesh,
    scratch_types=[
        pltpu.VMEM((MC,), jnp.float32),
        pltpu.VMEM((MC,), jnp.float32),
        pltpu.VMEM((MC,), jnp.float32),
        pltpu.VMEM((MC,), jnp.float32),
        pltpu.VMEM((MC,), jnp.float32),
    ],
)


def kernel(embedding, edge_index, edge_weight):
    row = edge_index[0]
    col = edge_index[1]
    zeros = jnp.zeros((RPT, DIM), jnp.float32)
    rowp, colp, wp, ncnk = _partition(row, col, edge_weight)
    x0 = embedding
    x1 = _layer(x0, rowp, colp, wp, ncnk, zeros)
    x2 = _layer(x1, rowp, colp, wp, ncnk, zeros)
    x3 = _layer(x2, rowp, colp, wp, ncnk, zeros)
    of = _mean(x0.reshape(-1), x1.reshape(-1), x2.reshape(-1), x3.reshape(-1))
    return of.reshape(N_NODES, DIM)
